# baseline (speedup 1.0000x reference)
import functools

import jax
import jax.numpy as jnp
from jax import lax
from jax.experimental import pallas as pl
from jax.experimental.pallas import tpu as pltpu
from jax.experimental.pallas import tpu_sc as plsc

_EMBED = 64
_NC, _NS = 2, 16
_NW = _NC * _NS
_CHUNK = 128
_SCALE = 8.0
_LANES = 16

_B = 4096 * 200
_CPW = _B // (_NW * _CHUNK)

_mesh = plsc.VectorSubcoreMesh(
    core_axis_name="c", subcore_axis_name="s",
    num_cores=_NC, num_subcores=_NS,
)


@functools.partial(
    pl.kernel,
    out_type=jax.ShapeDtypeStruct((_B, _EMBED), jnp.float32),
    mesh=_mesh,
    scratch_types=[
        pltpu.VMEM((_CPW, _CHUNK), jnp.int32),
        pltpu.VMEM((_CHUNK, _EMBED), jnp.float32),
        pltpu.SemaphoreType.DMA,
    ],
    compiler_params=pltpu.CompilerParams(use_tc_tiling_on_sc=False),
)
def _emb_lookup(x_hbm, table_hbm, out_hbm, idx_v, rows_v, gsem):
    wid = lax.axis_index("s") * _NC + lax.axis_index("c")
    pltpu.sync_copy(x_hbm.at[pl.ds(wid * _CPW, _CPW)], idx_v)
    out_base = wid * _CPW * _CHUNK

    def chunk_body(c, carry):
        pltpu.async_copy(table_hbm.at[idx_v.at[c]], rows_v, gsem).wait()

        def row_body(r, cc):
            for k in range(_EMBED // _LANES):
                sl = pl.ds(k * _LANES, _LANES)
                rows_v[r, sl] = rows_v[r, sl] * _SCALE
            return cc

        lax.fori_loop(0, _CHUNK, row_body, 0)
        pltpu.sync_copy(rows_v, out_hbm.at[pl.ds(out_base + c * _CHUNK, _CHUNK)])
        return carry

    lax.fori_loop(0, _CPW, chunk_body, 0)


def kernel(x, table):
    xf = x.astype(jnp.int32).reshape(_B // _CHUNK, _CHUNK)
    out = _emb_lookup(xf, table)
    return out.reshape(x.shape[0], x.shape[1], _EMBED)


# speedup vs baseline: 1.2072x; 1.2072x over previous
import functools

import jax
import jax.numpy as jnp
from jax import lax
from jax.experimental import pallas as pl
from jax.experimental.pallas import tpu as pltpu
from jax.experimental.pallas import tpu_sc as plsc

_EMBED = 64
_NC, _NS = 2, 16
_NW = _NC * _NS
_CHUNK = 128
_SCALE = 8.0
_LANES = 16
_NBUF = 4

_B = 4096 * 200
_CPW = _B // (_NW * _CHUNK)
_NGRP = _CPW // _NBUF

_mesh = plsc.VectorSubcoreMesh(
    core_axis_name="c", subcore_axis_name="s",
    num_cores=_NC, num_subcores=_NS,
)


@functools.partial(
    pl.kernel,
    out_type=jax.ShapeDtypeStruct((_B, _EMBED), jnp.float32),
    mesh=_mesh,
    scratch_types=[
        pltpu.VMEM((_CPW, _CHUNK), jnp.int32),
        pltpu.VMEM((_NBUF, _CHUNK, _EMBED), jnp.float32),
        pltpu.VMEM((_NBUF, _CHUNK, _EMBED), jnp.float32),
        pltpu.SemaphoreType.DMA((_NBUF,)),
        pltpu.SemaphoreType.DMA((_NBUF,)),
    ],
    compiler_params=pltpu.CompilerParams(use_tc_tiling_on_sc=False),
)
def _emb_lookup(x_hbm, table_hbm, out_hbm, idx_v, gbuf, wbuf, gsem, wsem):
    wid = lax.axis_index("s") * _NC + lax.axis_index("c")
    pltpu.sync_copy(x_hbm.at[pl.ds(wid * _CPW, _CPW)], idx_v)
    out_base = wid * _CPW * _CHUNK

    def fire_gather(c, b):
        pltpu.async_copy(table_hbm.at[idx_v.at[c]], gbuf.at[b], gsem.at[b])

    def wait_gather(c, b):
        pltpu.make_async_copy(table_hbm.at[idx_v.at[c]], gbuf.at[b],
                              gsem.at[b]).wait()

    def fire_write(c, b):
        pltpu.async_copy(wbuf.at[b],
                         out_hbm.at[pl.ds(out_base + c * _CHUNK, _CHUNK)],
                         wsem.at[b])

    def wait_write(c, b):
        pltpu.make_async_copy(wbuf.at[b],
                              out_hbm.at[pl.ds(out_base + c * _CHUNK, _CHUNK)],
                              wsem.at[b]).wait()

    def scale(b):
        @plsc.parallel_loop(0, _CHUNK, unroll=4)
        def _row(r):
            for k in range(_EMBED // _LANES):
                sl = pl.ds(k * _LANES, _LANES)
                wbuf[b, r, sl] = gbuf[b, r, sl] * _SCALE

    for b in range(_NBUF):
        fire_gather(b, b)

    for b in range(_NBUF):
        wait_gather(b, b)
        scale(b)
        fire_write(b, b)
        fire_gather(_NBUF + b, b)

    def group(g, carry):
        for b in range(_NBUF):
            c = g * _NBUF + b
            wait_gather(c, b)
            wait_write(c - _NBUF, b)
            scale(b)
            fire_write(c, b)
            fire_gather(c + _NBUF, b)
        return carry

    lax.fori_loop(1, _NGRP - 1, group, 0)

    for b in range(_NBUF):
        c = (_NGRP - 1) * _NBUF + b
        wait_gather(c, b)
        wait_write(c - _NBUF, b)
        scale(b)
        fire_write(c, b)

    for b in range(_NBUF):
        wait_write((_NGRP - 1) * _NBUF + b, b)


def kernel(x, table):
    xf = x.astype(jnp.int32).reshape(_B // _CHUNK, _CHUNK)
    out = _emb_lookup(xf, table)
    return out.reshape(x.shape[0], x.shape[1], _EMBED)
